# baseline (speedup 1.0000x reference)
import functools

import jax
import jax.numpy as jnp
from jax import lax
from jax.experimental import pallas as pl
from jax.experimental.pallas import tpu as pltpu
from jax.experimental.pallas import tpu_sc as plsc

_NUM_CORES = 2
_NUM_SUBCORES = 16
_NW = _NUM_CORES * _NUM_SUBCORES
_CH = 496
_NBUF = 2
_PREF = 2


def kernel(input_logits, labels, memory, labels_mem):
    batch, d = input_logits.shape
    k = memory.shape[0]
    tail = k - batch

    assert batch % (8 * _NW) == 0 and tail % 8 == 0
    b_per_w = batch // _NW
    t_per_w = -(-(tail // 8) // _NW) * 8
    n_in = -(-b_per_w // _CH)
    n_t = -(-t_per_w // _CH)

    mesh = plsc.VectorSubcoreMesh(core_axis_name="c", subcore_axis_name="s")

    @functools.partial(
        pl.kernel,
        mesh=mesh,
        out_type=(
            jax.ShapeDtypeStruct((k, d), memory.dtype),
            jax.ShapeDtypeStruct((k,), labels_mem.dtype),
        ),
        scratch_types=(
            [pltpu.VMEM((_CH, d), memory.dtype) for _ in range(_NBUF)]
            + [
                pltpu.VMEM((b_per_w,), labels.dtype),
                pltpu.VMEM((t_per_w,), labels_mem.dtype),
            ]
            + [pltpu.SemaphoreType.DMA for _ in range(2 * _NBUF + 2)]
        ),
    )
    def sk(in_hbm, lab_hbm, mem_hbm, labm_hbm, out_mem, out_lab, *scratch):
        bufs = scratch[:_NBUF]
        lv, tv = scratch[_NBUF], scratch[_NBUF + 1]
        gsems = scratch[_NBUF + 2 : 2 * _NBUF + 2]
        ssems = scratch[2 * _NBUF + 2 : 3 * _NBUF + 2]
        lsem_a, lsem_b = scratch[3 * _NBUF + 2], scratch[3 * _NBUF + 3]
        wid = lax.axis_index("s") * _NUM_CORES + lax.axis_index("c")

        ib = wid * b_per_w
        tb = jnp.minimum(batch + wid * t_per_w, k - t_per_w)
        tb = pl.multiple_of(tb, 8)

        chunks = []
        for i in range(n_in):
            s = jnp.minimum(ib + i * _CH, ib + b_per_w - _CH)
            chunks.append((in_hbm, pl.multiple_of(s, 8)))
        for i in range(n_t):
            s = jnp.minimum(tb + i * _CH, tb + t_per_w - _CH)
            chunks.append((mem_hbm, pl.multiple_of(s, 8)))
        n = len(chunks)

        g = [None] * n
        s_ = [None] * n

        def issue_gather(j):
            if j - _NBUF >= 0:
                s_[j - _NBUF].wait()
            src, st = chunks[j]
            b = j % _NBUF
            g[j] = pltpu.async_copy(src.at[pl.ds(st, _CH)], bufs[b], gsems[b])

        for j in range(min(_PREF, n)):
            issue_gather(j)
        for i in range(n):
            b = i % _NBUF
            g[i].wait()
            s_[i] = pltpu.async_copy(
                bufs[b], out_mem.at[pl.ds(chunks[i][1], _CH)], ssems[b]
            )
            if i + _PREF < n:
                issue_gather(i + _PREF)

        c0 = pltpu.async_copy(lab_hbm.at[pl.ds(ib, b_per_w)], lv, lsem_a)
        c1 = pltpu.async_copy(labm_hbm.at[pl.ds(tb, t_per_w)], tv, lsem_b)
        c0.wait()
        c2 = pltpu.async_copy(lv, out_lab.at[pl.ds(ib, b_per_w)], lsem_a)
        c1.wait()
        c3 = pltpu.async_copy(tv, out_lab.at[pl.ds(tb, t_per_w)], lsem_b)
        c2.wait()
        c3.wait()

        for i in range(max(0, n - _NBUF), n):
            s_[i].wait()

    new_memory, new_labels_mem = sk(input_logits, labels, memory, labels_mem)
    return (new_memory, new_labels_mem, jnp.array(batch % k, dtype=jnp.int32))


# speedup vs baseline: 1.1038x; 1.1038x over previous
import functools

import jax
import jax.numpy as jnp
from jax import lax
from jax.experimental import pallas as pl
from jax.experimental.pallas import tpu as pltpu
from jax.experimental.pallas import tpu_sc as plsc

_NUM_CORES = 2
_NUM_SUBCORES = 16
_NW = _NUM_CORES * _NUM_SUBCORES
_CH = 160
_NBUF = 6
_PREF = 3


def kernel(input_logits, labels, memory, labels_mem):
    batch, d = input_logits.shape
    k = memory.shape[0]
    tail = k - batch

    assert batch % (8 * _NW) == 0 and tail % 8 == 0
    b_per_w = batch // _NW
    t_per_w = -(-(tail // 8) // _NW) * 8
    n_in = -(-b_per_w // _CH)
    n_t = -(-t_per_w // _CH)

    mesh = plsc.VectorSubcoreMesh(core_axis_name="c", subcore_axis_name="s")

    @functools.partial(
        pl.kernel,
        mesh=mesh,
        out_type=(
            jax.ShapeDtypeStruct((k, d), memory.dtype),
            jax.ShapeDtypeStruct((k,), labels_mem.dtype),
        ),
        scratch_types=(
            [pltpu.VMEM((_CH, d), memory.dtype) for _ in range(_NBUF)]
            + [
                pltpu.VMEM((b_per_w,), labels.dtype),
                pltpu.VMEM((t_per_w,), labels_mem.dtype),
            ]
            + [pltpu.SemaphoreType.DMA for _ in range(2 * _NBUF + 2)]
        ),
    )
    def sk(in_hbm, lab_hbm, mem_hbm, labm_hbm, out_mem, out_lab, *scratch):
        bufs = scratch[:_NBUF]
        lv, tv = scratch[_NBUF], scratch[_NBUF + 1]
        gsems = scratch[_NBUF + 2 : 2 * _NBUF + 2]
        ssems = scratch[2 * _NBUF + 2 : 3 * _NBUF + 2]
        lsem_a, lsem_b = scratch[3 * _NBUF + 2], scratch[3 * _NBUF + 3]
        wid = lax.axis_index("s") * _NUM_CORES + lax.axis_index("c")

        ib = wid * b_per_w
        tb = jnp.minimum(batch + wid * t_per_w, k - t_per_w)
        tb = pl.multiple_of(tb, 8)

        chunks = []
        for i in range(n_in):
            s = jnp.minimum(ib + i * _CH, ib + b_per_w - _CH)
            chunks.append((in_hbm, pl.multiple_of(s, 8)))
        for i in range(n_t):
            s = jnp.minimum(tb + i * _CH, tb + t_per_w - _CH)
            chunks.append((mem_hbm, pl.multiple_of(s, 8)))
        n = len(chunks)

        g = [None] * n
        s_ = [None] * n

        def issue_gather(j):
            if j - _NBUF >= 0:
                s_[j - _NBUF].wait()
            src, st = chunks[j]
            b = j % _NBUF
            g[j] = pltpu.async_copy(src.at[pl.ds(st, _CH)], bufs[b], gsems[b])

        for j in range(min(_PREF, n)):
            issue_gather(j)
        for i in range(n):
            b = i % _NBUF
            g[i].wait()
            s_[i] = pltpu.async_copy(
                bufs[b], out_mem.at[pl.ds(chunks[i][1], _CH)], ssems[b]
            )
            if i + _PREF < n:
                issue_gather(i + _PREF)

        c0 = pltpu.async_copy(lab_hbm.at[pl.ds(ib, b_per_w)], lv, lsem_a)
        c1 = pltpu.async_copy(labm_hbm.at[pl.ds(tb, t_per_w)], tv, lsem_b)
        c0.wait()
        c2 = pltpu.async_copy(lv, out_lab.at[pl.ds(ib, b_per_w)], lsem_a)
        c1.wait()
        c3 = pltpu.async_copy(tv, out_lab.at[pl.ds(tb, t_per_w)], lsem_b)
        c2.wait()
        c3.wait()

        for i in range(max(0, n - _NBUF), n):
            s_[i].wait()

    new_memory, new_labels_mem = sk(input_logits, labels, memory, labels_mem)
    return (new_memory, new_labels_mem, jnp.array(batch % k, dtype=jnp.int32))


# speedup vs baseline: 1.1181x; 1.0130x over previous
import functools

import jax
import jax.numpy as jnp
from jax import lax
from jax.experimental import pallas as pl
from jax.experimental.pallas import tpu as pltpu
from jax.experimental.pallas import tpu_sc as plsc

_NUM_CORES = 2
_NUM_SUBCORES = 16
_NW = _NUM_CORES * _NUM_SUBCORES
_CH = 120
_NBUF = 8
_PREF = 4


def kernel(input_logits, labels, memory, labels_mem):
    batch, d = input_logits.shape
    k = memory.shape[0]
    tail = k - batch

    assert batch % (8 * _NW) == 0 and tail % 8 == 0
    b_per_w = batch // _NW
    t_per_w = -(-(tail // 8) // _NW) * 8
    n_in = -(-b_per_w // _CH)
    n_t = -(-t_per_w // _CH)

    mesh = plsc.VectorSubcoreMesh(core_axis_name="c", subcore_axis_name="s")

    @functools.partial(
        pl.kernel,
        mesh=mesh,
        out_type=(
            jax.ShapeDtypeStruct((k, d), memory.dtype),
            jax.ShapeDtypeStruct((k,), labels_mem.dtype),
        ),
        scratch_types=(
            [pltpu.VMEM((_CH, d), memory.dtype) for _ in range(_NBUF)]
            + [
                pltpu.VMEM((b_per_w,), labels.dtype),
                pltpu.VMEM((t_per_w,), labels_mem.dtype),
            ]
            + [pltpu.SemaphoreType.DMA for _ in range(2 * _NBUF + 2)]
        ),
    )
    def sk(in_hbm, lab_hbm, mem_hbm, labm_hbm, out_mem, out_lab, *scratch):
        bufs = scratch[:_NBUF]
        lv, tv = scratch[_NBUF], scratch[_NBUF + 1]
        gsems = scratch[_NBUF + 2 : 2 * _NBUF + 2]
        ssems = scratch[2 * _NBUF + 2 : 3 * _NBUF + 2]
        lsem_a, lsem_b = scratch[3 * _NBUF + 2], scratch[3 * _NBUF + 3]
        wid = lax.axis_index("s") * _NUM_CORES + lax.axis_index("c")

        ib = wid * b_per_w
        tb = jnp.minimum(batch + wid * t_per_w, k - t_per_w)
        tb = pl.multiple_of(tb, 8)

        chunks = []
        for i in range(n_in):
            s = jnp.minimum(ib + i * _CH, ib + b_per_w - _CH)
            chunks.append((in_hbm, pl.multiple_of(s, 8)))
        for i in range(n_t):
            s = jnp.minimum(tb + i * _CH, tb + t_per_w - _CH)
            chunks.append((mem_hbm, pl.multiple_of(s, 8)))
        n = len(chunks)

        g = [None] * n
        s_ = [None] * n

        def issue_gather(j):
            if j - _NBUF >= 0:
                s_[j - _NBUF].wait()
            src, st = chunks[j]
            b = j % _NBUF
            g[j] = pltpu.async_copy(src.at[pl.ds(st, _CH)], bufs[b], gsems[b])

        for j in range(min(_PREF, n)):
            issue_gather(j)
        for i in range(n):
            b = i % _NBUF
            g[i].wait()
            s_[i] = pltpu.async_copy(
                bufs[b], out_mem.at[pl.ds(chunks[i][1], _CH)], ssems[b]
            )
            if i + _PREF < n:
                issue_gather(i + _PREF)

        c0 = pltpu.async_copy(lab_hbm.at[pl.ds(ib, b_per_w)], lv, lsem_a)
        c1 = pltpu.async_copy(labm_hbm.at[pl.ds(tb, t_per_w)], tv, lsem_b)
        c0.wait()
        c2 = pltpu.async_copy(lv, out_lab.at[pl.ds(ib, b_per_w)], lsem_a)
        c1.wait()
        c3 = pltpu.async_copy(tv, out_lab.at[pl.ds(tb, t_per_w)], lsem_b)
        c2.wait()
        c3.wait()

        for i in range(max(0, n - _NBUF), n):
            s_[i].wait()

    new_memory, new_labels_mem = sk(input_logits, labels, memory, labels_mem)
    return (new_memory, new_labels_mem, jnp.array(batch % k, dtype=jnp.int32))
